# baseline (speedup 1.0000x reference)
import functools

import jax
import jax.numpy as jnp
from jax import lax
from jax.experimental import pallas as pl
from jax.experimental.pallas import tpu as pltpu
from jax.experimental.pallas import tpu_sc as plsc

_NUM_CORES = 2
_NUM_SUBCORES = 16
_NUM_WORKERS = _NUM_CORES * _NUM_SUBCORES
_LANES = 16


@functools.lru_cache(maxsize=None)
def _build(B, PL_LEN, V, D):
    assert B % (_NUM_WORKERS * _LANES) == 0
    bpw = B // _NUM_WORKERS
    nblk = bpw // _LANES
    assert D == _LANES

    mesh = plsc.VectorSubcoreMesh(
        core_axis_name="c", subcore_axis_name="s",
        num_cores=_NUM_CORES, num_subcores=_NUM_SUBCORES)

    @functools.partial(
        pl.kernel,
        out_type=jax.ShapeDtypeStruct((B * D,), jnp.float32),
        mesh=mesh,
        compiler_params=pltpu.CompilerParams(needs_layout_passes=False),
        scratch_types=[
            pltpu.VMEM((bpw * PL_LEN,), jnp.int32),
            pltpu.VMEM((V * D,), jnp.float32),
            pltpu.VMEM((bpw * D,), jnp.float32),
        ],
    )
    def plate_encode(idx_hbm, table_hbm, out_hbm, idx_v, table_v, out_v):
        wid = lax.axis_index("s") * _NUM_CORES + lax.axis_index("c")
        base = wid * bpw
        pltpu.sync_copy(idx_hbm.at[pl.ds(base * PL_LEN, bpw * PL_LEN)], idx_v)
        pltpu.sync_copy(table_hbm, table_v)

        lane = lax.broadcasted_iota(jnp.int32, (_LANES,), 0)
        lane_p = lane * PL_LEN
        lane_d = lane * D
        scale = jnp.float32(1.0 / PL_LEN)

        def block(b, carry):
            ivs = [plsc.load_gather(idx_v, [lane_p + (b * (_LANES * PL_LEN) + c)])
                   for c in range(PL_LEN)]
            rows = [iv * D for iv in ivs]
            for d in range(D):
                acc = plsc.load_gather(table_v, [rows[0] + d])
                for c in range(1, PL_LEN):
                    acc = acc + plsc.load_gather(table_v, [rows[c] + d])
                plsc.store_scatter(out_v, [lane_d + (b * (_LANES * D) + d)],
                                   acc * scale)
            return carry

        lax.fori_loop(0, nblk, block, 0, unroll=4)
        pltpu.sync_copy(out_v, out_hbm.at[pl.ds(base * D, bpw * D)])

    return plate_encode


def kernel(plates_indices, embedding_table):
    B, PL_LEN = plates_indices.shape
    V, D = embedding_table.shape
    fn = _build(B, PL_LEN, V, D)
    out_flat = fn(plates_indices.astype(jnp.int32).reshape(B * PL_LEN),
                  embedding_table.astype(jnp.float32).reshape(V * D))
    return out_flat.reshape(B, D)


# speedup vs baseline: 1.3499x; 1.3499x over previous
import functools

import jax
import jax.numpy as jnp
from jax import lax
from jax.experimental import pallas as pl
from jax.experimental.pallas import tpu as pltpu
from jax.experimental.pallas import tpu_sc as plsc

_NUM_CORES = 2
_NUM_SUBCORES = 16
_NUM_WORKERS = _NUM_CORES * _NUM_SUBCORES
_LANES = 16


@functools.lru_cache(maxsize=None)
def _build(B, PL_LEN, V, D):
    assert B % (_NUM_WORKERS * _LANES) == 0
    bpw = B // _NUM_WORKERS
    nblk = bpw // _LANES
    assert D == _LANES

    mesh = plsc.VectorSubcoreMesh(
        core_axis_name="c", subcore_axis_name="s",
        num_cores=_NUM_CORES, num_subcores=_NUM_SUBCORES)

    @functools.partial(
        pl.kernel,
        out_type=jax.ShapeDtypeStruct((B * D,), jnp.float32),
        mesh=mesh,
        compiler_params=pltpu.CompilerParams(needs_layout_passes=False),
        scratch_types=[
            pltpu.VMEM((bpw * PL_LEN,), jnp.int32),
            pltpu.VMEM((V * D,), jnp.float32),
            pltpu.VMEM((bpw * D,), jnp.float32),
        ],
    )
    def plate_encode(idx_hbm, table_hbm, out_hbm, idx_v, table_v, out_v):
        wid = lax.axis_index("s") * _NUM_CORES + lax.axis_index("c")
        base = wid * bpw
        pltpu.sync_copy(idx_hbm.at[pl.ds(base * PL_LEN, bpw * PL_LEN)], idx_v)
        pltpu.sync_copy(table_hbm, table_v)

        lane = lax.broadcasted_iota(jnp.int32, (_LANES,), 0)
        lane_p = lane * PL_LEN
        lane_d = lane * D
        scale = jnp.float32(1.0 / PL_LEN)

        def block(b, carry):
            ivs = [plsc.load_gather(idx_v, [lane_p + (b * (_LANES * PL_LEN) + c)])
                   for c in range(PL_LEN)]
            for d in range(D):
                acc = plsc.load_gather(table_v, [ivs[0] + (d * V)])
                for c in range(1, PL_LEN):
                    acc = acc + plsc.load_gather(table_v, [ivs[c] + (d * V)])
                plsc.store_scatter(out_v, [lane_d + (b * (_LANES * D) + d)],
                                   acc * scale)
            return carry

        lax.fori_loop(0, nblk, block, 0, unroll=False)
        pltpu.sync_copy(out_v, out_hbm.at[pl.ds(base * D, bpw * D)])

    return plate_encode


def kernel(plates_indices, embedding_table):
    B, PL_LEN = plates_indices.shape
    V, D = embedding_table.shape
    fn = _build(B, PL_LEN, V, D)
    out_flat = fn(plates_indices.astype(jnp.int32).reshape(B * PL_LEN),
                  embedding_table.astype(jnp.float32).T.reshape(V * D))
    return out_flat.reshape(B, D)
